# baseline (speedup 1.0000x reference)
import functools

import jax
import jax.numpy as jnp
from jax import lax
from jax.experimental import pallas as pl
from jax.experimental.pallas import tpu as pltpu
from jax.experimental.pallas import tpu_sc as plsc

N = 10000
D_IN = 128
H = 64
L = 8
C = 10

NC = 2
HC = H // NC
NT = 16
RP = 640
NP = NT * RP
E = 320000
EC = 128
ET = 160
EPT = ET * EC
EP = NT * EPT
RSUB = 128
GC = 16


def _fori(lo, hi, f):
    lax.fori_loop(lo, hi, lambda i, c: (f(i), c)[1], None)


def _sc_body(h0_hbm, src_hbm, dst_hbm, pai_hbm, out_hbm,
             a_sp, b_sp, out_sp,
             esrc_t, edst_t, rows0, rows1,
             bslab, oslab, dinv_t, pai_t,
             gsem0, gsem1, ssem0):
    rowbufs = (rows0, rows1)
    gsems = (gsem0, gsem1)
    cid = lax.axis_index("c")
    tid = lax.axis_index("s")
    rbase = tid * RP

    pltpu.sync_copy(pai_hbm, pai_t)
    pltpu.sync_copy(src_hbm.at[pl.ds(tid * ET, ET)], esrc_t)
    pltpu.sync_copy(dst_hbm.at[pl.ds(tid * ET, ET)], edst_t)

    def fill_ones(i):
        for c in range(HC // 16):
            rows0[i, pl.ds(c * 16, 16)] = jnp.full((16,), 1.0,
                                                   dtype=jnp.float32)
    _fori(0, EC, fill_ones)

    def init_deg(s):
        pltpu.sync_copy(rows0, b_sp.at[pl.ds(rbase + s * EC, EC)])
    _fori(0, RP // EC, init_deg)

    plsc.subcore_barrier()

    def deg_grp(g):
        cb = g * GC
        sds = [pltpu.async_copy(rows0, b_sp.at[edst_t.at[cb + j]], ssem0,
                                add=True)
               for j in range(GC)]
        for d in sds:
            d.wait()
    _fori(0, ET // GC, deg_grp)

    def h0_sub(s):
        base = rbase + s * RSUB
        pltpu.sync_copy(h0_hbm.at[pl.ds(base, RSUB), cid],
                        a_sp.at[pl.ds(base, RSUB)])
    _fori(0, RP // RSUB, h0_sub)

    plsc.subcore_barrier()

    pai0 = pai_t[0, :]

    def init_sub(s):
        base = rbase + s * RSUB
        pltpu.sync_copy(b_sp.at[pl.ds(base, RSUB)], bslab)
        pltpu.sync_copy(a_sp.at[pl.ds(base, RSUB)], oslab)

        def rowf(r):
            deg = bslab[r, pl.ds(0, 16)]
            iv = lax.bitcast_convert_type(deg, jnp.int32)
            y = lax.bitcast_convert_type(
                jnp.full((16,), 0x5F3759DF, dtype=jnp.int32) - (iv >> 1),
                jnp.float32)
            y = y * (1.5 - 0.5 * deg * y * y)
            y = y * (1.5 - 0.5 * deg * y * y)
            y = y * (1.5 - 0.5 * deg * y * y)
            dinv_t[s * RSUB + r, :] = y
            for c in range(HC // 16):
                sl = pl.ds(c * 16, 16)
                v = oslab[r, sl] * y
                bslab[r, sl] = v
                oslab[r, sl] = v * pai0
        _fori(0, RSUB, rowf)
        pltpu.sync_copy(bslab, a_sp.at[pl.ds(base, RSUB)])
        pltpu.sync_copy(bslab, b_sp.at[pl.ds(base, RSUB)])
        pltpu.sync_copy(oslab, out_sp.at[pl.ds(base, RSUB)])
    _fori(0, RP // RSUB, init_sub)

    def layer(l):
        plsc.subcore_barrier()

        dummy = h0_hbm.at[pl.ds(0, EC), cid]
        pltpu.async_copy(a_sp.at[esrc_t.at[0]], rows0, gsem0)

        def edge_grp(g):
            cb = g * GC
            gds = [None] * (GC + 1)
            for j in range(GC):
                b = j % 2
                nb = (j + 1) % 2
                nxt = (cb + j + 1) if j < GC - 1 else lax.rem(cb + GC, ET)
                gds[j + 1] = pltpu.async_copy(
                    a_sp.at[esrc_t.at[nxt]], rowbufs[nb], gsems[nb])
                if j == 0:
                    pltpu.make_async_copy(dummy, rows0, gsem0).wait()
                else:
                    gds[j].wait()
                pltpu.sync_copy(rowbufs[b], b_sp.at[edst_t.at[cb + j]],
                                add=True)
        _fori(0, ET // GC, edge_grp)
        pltpu.make_async_copy(dummy, rows0, gsem0).wait()

        plsc.subcore_barrier()

        pai_l = pai_t[l + 1, :]

        def pass_sub(s):
            base = rbase + s * RSUB
            pltpu.sync_copy(b_sp.at[pl.ds(base, RSUB)], bslab)
            pltpu.sync_copy(out_sp.at[pl.ds(base, RSUB)], oslab)

            def rowf(r):
                dv = dinv_t[s * RSUB + r, :]
                d2v = dv * dv
                for c in range(HC // 16):
                    sl = pl.ds(c * 16, 16)
                    g = jnp.maximum(bslab[r, sl] * d2v, 0.0)
                    bslab[r, sl] = g
                    oslab[r, sl] = oslab[r, sl] + pai_l * g
            _fori(0, RSUB, rowf)
            pltpu.sync_copy(bslab, a_sp.at[pl.ds(base, RSUB)])
            pltpu.sync_copy(bslab, b_sp.at[pl.ds(base, RSUB)])
            pltpu.sync_copy(oslab, out_sp.at[pl.ds(base, RSUB)])
        _fori(0, RP // RSUB, pass_sub)
    _fori(0, L, layer)

    def fin_sub(s):
        base = rbase + s * RSUB
        pltpu.sync_copy(out_sp.at[pl.ds(base, RSUB)], oslab)

        def rowf(r):
            dv = dinv_t[s * RSUB + r, :]
            for c in range(HC // 16):
                sl = pl.ds(c * 16, 16)
                oslab[r, sl] = oslab[r, sl] / dv
        _fori(0, RSUB, rowf)
        pltpu.sync_copy(oslab, out_hbm.at[pl.ds(base, RSUB), cid])
    _fori(0, RP // RSUB, fin_sub)


@jax.jit
def _sc_propagate(h0s, src2d, dst2d, pai16):
    mesh = plsc.VectorSubcoreMesh(core_axis_name="c", subcore_axis_name="s")
    f = pl.kernel(
        _sc_body,
        out_type=jax.ShapeDtypeStruct((NP, NC, HC), jnp.float32),
        mesh=mesh,
        compiler_params=pltpu.CompilerParams(use_tc_tiling_on_sc=False),
        scratch_types=[
            pltpu.VMEM_SHARED((NP, HC), jnp.float32),
            pltpu.VMEM_SHARED((NP, HC), jnp.float32),
            pltpu.VMEM_SHARED((NP, HC), jnp.float32),
            pltpu.VMEM((ET, EC), jnp.int32),
            pltpu.VMEM((ET, EC), jnp.int32),
            pltpu.VMEM((EC, HC), jnp.float32),
            pltpu.VMEM((EC, HC), jnp.float32),
            pltpu.VMEM((RSUB, HC), jnp.float32),
            pltpu.VMEM((RSUB, HC), jnp.float32),
            pltpu.VMEM((RP, 16), jnp.float32),
            pltpu.VMEM((16, 16), jnp.float32),
            pltpu.SemaphoreType.DMA,
            pltpu.SemaphoreType.DMA,
            pltpu.SemaphoreType.DMA,
        ],
    )
    return f(h0s, src2d, dst2d, pai16)


def _mm_relu_body(x_ref, w_ref, b_ref, o_ref):
    acc = jnp.dot(x_ref[...], w_ref[...], preferred_element_type=jnp.float32)
    o_ref[...] = jnp.maximum(acc + b_ref[...], 0.0)


def _head_body(o_ref, w_ref, b_ref, l_ref):
    logits = jnp.dot(o_ref[...], w_ref[...],
                     preferred_element_type=jnp.float32) + b_ref[...]
    m = jnp.max(logits, axis=-1, keepdims=True)
    z = logits - m
    lse = jnp.log(jnp.sum(jnp.exp(z), axis=-1, keepdims=True))
    l_ref[...] = z - lse


def kernel(x, edge_index, W0, b0, Wconv, W1, b1, pai):
    del Wconv
    src = edge_index[0]
    dst = edge_index[1]
    pad_e = EP - E
    fill = jnp.full((pad_e,), NP - 1, dtype=src.dtype)
    src2d = jnp.concatenate([src, fill]).reshape(NT * ET, EC)
    dst2d = jnp.concatenate([dst, fill]).reshape(NT * ET, EC)
    pai16 = jnp.tile(jnp.pad(pai[0], (0, 16 - (L + 1))).reshape(16, 1),
                     (1, 16))

    h0 = pl.pallas_call(
        _mm_relu_body,
        grid=(10,),
        in_specs=[
            pl.BlockSpec((N // 10, D_IN), lambda i: (i, 0)),
            pl.BlockSpec((D_IN, H), lambda i: (0, 0)),
            pl.BlockSpec((1, H), lambda i: (0, 0)),
        ],
        out_specs=pl.BlockSpec((N // 10, H), lambda i: (i, 0)),
        out_shape=jax.ShapeDtypeStruct((N, H), jnp.float32),
    )(x, W0, b0.reshape(1, H))
    h0s = jnp.pad(h0, ((0, NP - N), (0, 0))).reshape(NP, NC, HC)

    outg = _sc_propagate(h0s, src2d, dst2d, pai16)
    outf = outg.reshape(NP, H)

    W1p = jnp.pad(W1, ((0, 0), (0, 128 - C)))
    b1p = jnp.pad(b1, (0, 128 - C), constant_values=-1e30).reshape(1, 128)
    ls = pl.pallas_call(
        _head_body,
        grid=(10,),
        in_specs=[
            pl.BlockSpec((N // 10, H), lambda i: (i, 0)),
            pl.BlockSpec((H, 128), lambda i: (0, 0)),
            pl.BlockSpec((1, 128), lambda i: (0, 0)),
        ],
        out_specs=pl.BlockSpec((N // 10, 128), lambda i: (i, 0)),
        out_shape=jax.ShapeDtypeStruct((N, 128), jnp.float32),
    )(outf[:N], W1p, b1p)
    return ls[:, :C]


# speedup vs baseline: 1.1186x; 1.1186x over previous
import functools

import jax
import jax.numpy as jnp
from jax import lax
from jax.experimental import pallas as pl
from jax.experimental.pallas import tpu as pltpu
from jax.experimental.pallas import tpu_sc as plsc

N = 10000
D_IN = 128
H = 64
L = 8
C = 10

NC = 2
HC = H // NC
NT = 16
RP = 640
NP = NT * RP
E = 320000
EC = 128
ET = 160
EPT = ET * EC
EP = NT * EPT
RSUB = 128
GC = 16


def _fori(lo, hi, f):
    lax.fori_loop(lo, hi, lambda i, c: (f(i), c)[1], None)


def _sc_body(h0_hbm, src_hbm, dst_hbm, pai_hbm, out_hbm,
             a_sp, b_sp, out_sp,
             esrc_t, edst_t, rows0, rows1, rows2, rows3,
             dinv_t, pai_t,
             gsem0, gsem1, gsem2, gsem3, ssem0, ssem1, ssem2, ssem3):
    rowbufs = (rows0, rows1, rows2, rows3)
    gsems = (gsem0, gsem1, gsem2, gsem3)
    ssems = (ssem0, ssem1, ssem2, ssem3)
    bslab = rows0
    oslab = rows1
    cid = lax.axis_index("c")
    tid = lax.axis_index("s")
    rbase = tid * RP

    pltpu.sync_copy(pai_hbm, pai_t)
    pltpu.sync_copy(src_hbm.at[pl.ds(tid * ET, ET)], esrc_t)
    pltpu.sync_copy(dst_hbm.at[pl.ds(tid * ET, ET)], edst_t)

    def fill_ones(i):
        for c in range(HC // 16):
            rows0[i, pl.ds(c * 16, 16)] = jnp.full((16,), 1.0,
                                                   dtype=jnp.float32)
    _fori(0, EC, fill_ones)

    def init_deg(s):
        pltpu.sync_copy(rows0, b_sp.at[pl.ds(rbase + s * EC, EC)])
    _fori(0, RP // EC, init_deg)

    plsc.subcore_barrier()

    def deg_grp(g):
        cb = g * GC
        sds = [pltpu.async_copy(rows0, b_sp.at[edst_t.at[cb + j]], ssem0,
                                add=True)
               for j in range(GC)]
        for d in sds:
            d.wait()
    _fori(0, ET // GC, deg_grp)

    def h0_sub(s):
        base = rbase + s * RSUB
        pltpu.sync_copy(h0_hbm.at[cid, pl.ds(base, RSUB)],
                        a_sp.at[pl.ds(base, RSUB)])
    _fori(0, RP // RSUB, h0_sub)

    plsc.subcore_barrier()

    pai0 = pai_t[0, :]

    def init_sub(s):
        base = rbase + s * RSUB
        pltpu.sync_copy(b_sp.at[pl.ds(base, RSUB)], bslab)
        pltpu.sync_copy(a_sp.at[pl.ds(base, RSUB)], oslab)

        def rowf(r):
            deg = bslab[r, pl.ds(0, 16)]
            iv = lax.bitcast_convert_type(deg, jnp.int32)
            y = lax.bitcast_convert_type(
                jnp.full((16,), 0x5F3759DF, dtype=jnp.int32) - (iv >> 1),
                jnp.float32)
            y = y * (1.5 - 0.5 * deg * y * y)
            y = y * (1.5 - 0.5 * deg * y * y)
            y = y * (1.5 - 0.5 * deg * y * y)
            dinv_t[s * RSUB + r, :] = y
            for c in range(HC // 16):
                sl = pl.ds(c * 16, 16)
                v = oslab[r, sl] * y
                bslab[r, sl] = v
                oslab[r, sl] = v * pai0
        _fori(0, RSUB, rowf)
        pltpu.sync_copy(bslab, a_sp.at[pl.ds(base, RSUB)])
        pltpu.sync_copy(bslab, b_sp.at[pl.ds(base, RSUB)])
        pltpu.sync_copy(oslab, out_sp.at[pl.ds(base, RSUB)])
    _fori(0, RP // RSUB, init_sub)

    def layer(l):
        plsc.subcore_barrier()

        dummy = h0_hbm.at[cid, pl.ds(0, EC)]
        for b in range(1, 4):
            pltpu.async_copy(rowbufs[b], out_hbm.at[cid, pl.ds(rbase, EC)],
                             ssems[b])
        pltpu.async_copy(a_sp.at[esrc_t.at[0]], rows0, gsem0)

        def edge_grp(g):
            cb = g * GC
            gds = [None] * (GC + 1)
            for j in range(GC):
                b = j % 4
                nb = (j + 1) % 4
                pltpu.make_async_copy(dummy, rowbufs[nb], ssems[nb]).wait()
                nxt = (cb + j + 1) if j < GC - 1 else lax.rem(cb + GC, ET)
                gds[j + 1] = pltpu.async_copy(
                    a_sp.at[esrc_t.at[nxt]], rowbufs[nb], gsems[nb])
                if j == 0:
                    pltpu.make_async_copy(dummy, rows0, gsem0).wait()
                else:
                    gds[j].wait()
                pltpu.async_copy(rowbufs[b], b_sp.at[edst_t.at[cb + j]],
                                 ssems[b], add=True)
        _fori(0, ET // GC, edge_grp)
        pltpu.make_async_copy(dummy, rows0, gsem0).wait()
        for b in range(1, 4):
            pltpu.make_async_copy(dummy, rowbufs[b], ssems[b]).wait()

        plsc.subcore_barrier()

        pai_l = pai_t[l + 1, :]

        def pass_sub(s):
            base = rbase + s * RSUB
            pltpu.sync_copy(b_sp.at[pl.ds(base, RSUB)], bslab)
            pltpu.sync_copy(out_sp.at[pl.ds(base, RSUB)], oslab)

            def rowf(r):
                dv = dinv_t[s * RSUB + r, :]
                d2v = dv * dv
                for c in range(HC // 16):
                    sl = pl.ds(c * 16, 16)
                    g = jnp.maximum(bslab[r, sl] * d2v, 0.0)
                    bslab[r, sl] = g
                    oslab[r, sl] = oslab[r, sl] + pai_l * g
            _fori(0, RSUB, rowf)
            pltpu.sync_copy(bslab, a_sp.at[pl.ds(base, RSUB)])
            pltpu.sync_copy(bslab, b_sp.at[pl.ds(base, RSUB)])
            pltpu.sync_copy(oslab, out_sp.at[pl.ds(base, RSUB)])
        _fori(0, RP // RSUB, pass_sub)
    _fori(0, L, layer)

    def fin_sub(s):
        base = rbase + s * RSUB
        pltpu.sync_copy(out_sp.at[pl.ds(base, RSUB)], oslab)

        def rowf(r):
            dv = dinv_t[s * RSUB + r, :]
            for c in range(HC // 16):
                sl = pl.ds(c * 16, 16)
                oslab[r, sl] = oslab[r, sl] / dv
        _fori(0, RSUB, rowf)
        pltpu.sync_copy(oslab, out_hbm.at[cid, pl.ds(base, RSUB)])
    _fori(0, RP // RSUB, fin_sub)


@jax.jit
def _sc_propagate(h0s, src2d, dst2d, pai16):
    mesh = plsc.VectorSubcoreMesh(core_axis_name="c", subcore_axis_name="s")
    f = pl.kernel(
        _sc_body,
        out_type=jax.ShapeDtypeStruct((NC, NP, HC), jnp.float32),
        mesh=mesh,
        compiler_params=pltpu.CompilerParams(use_tc_tiling_on_sc=False),
        scratch_types=[
            pltpu.VMEM_SHARED((NP, HC), jnp.float32),
            pltpu.VMEM_SHARED((NP, HC), jnp.float32),
            pltpu.VMEM_SHARED((NP, HC), jnp.float32),
            pltpu.VMEM((ET, EC), jnp.int32),
            pltpu.VMEM((ET, EC), jnp.int32),
            pltpu.VMEM((EC, HC), jnp.float32),
            pltpu.VMEM((EC, HC), jnp.float32),
            pltpu.VMEM((EC, HC), jnp.float32),
            pltpu.VMEM((EC, HC), jnp.float32),
            pltpu.VMEM((RP, 16), jnp.float32),
            pltpu.VMEM((16, 16), jnp.float32),
            pltpu.SemaphoreType.DMA,
            pltpu.SemaphoreType.DMA,
            pltpu.SemaphoreType.DMA,
            pltpu.SemaphoreType.DMA,
            pltpu.SemaphoreType.DMA,
            pltpu.SemaphoreType.DMA,
            pltpu.SemaphoreType.DMA,
            pltpu.SemaphoreType.DMA,
        ],
    )
    return f(h0s, src2d, dst2d, pai16)


def _mm_relu_body(x_ref, w_ref, b_ref, o_ref):
    acc = jnp.dot(x_ref[...], w_ref[...], preferred_element_type=jnp.float32)
    o_ref[...] = jnp.maximum(acc + b_ref[...], 0.0)


def _head_body(o_ref, w_ref, b_ref, l_ref):
    logits = jnp.dot(o_ref[...], w_ref[...],
                     preferred_element_type=jnp.float32) + b_ref[...]
    m = jnp.max(logits, axis=-1, keepdims=True)
    z = logits - m
    lse = jnp.log(jnp.sum(jnp.exp(z), axis=-1, keepdims=True))
    l_ref[...] = z - lse


def kernel(x, edge_index, W0, b0, Wconv, W1, b1, pai):
    del Wconv
    src = edge_index[0]
    dst = edge_index[1]
    pad_e = EP - E
    fill = jnp.full((pad_e,), NP - 1, dtype=src.dtype)
    src2d = jnp.concatenate([src, fill]).reshape(NT * ET, EC)
    dst2d = jnp.concatenate([dst, fill]).reshape(NT * ET, EC)
    pai16 = jnp.tile(jnp.pad(pai[0], (0, 16 - (L + 1))).reshape(16, 1),
                     (1, 16))

    h0 = pl.pallas_call(
        _mm_relu_body,
        grid=(10,),
        in_specs=[
            pl.BlockSpec((N // 10, D_IN), lambda i: (i, 0)),
            pl.BlockSpec((D_IN, H), lambda i: (0, 0)),
            pl.BlockSpec((1, H), lambda i: (0, 0)),
        ],
        out_specs=pl.BlockSpec((N // 10, H), lambda i: (i, 0)),
        out_shape=jax.ShapeDtypeStruct((N, H), jnp.float32),
    )(x, W0, b0.reshape(1, H))
    h0s = jnp.pad(h0, ((0, NP - N), (0, 0))).reshape(NP, NC, HC)
    h0s = h0s.transpose(1, 0, 2)

    outg = _sc_propagate(h0s, src2d, dst2d, pai16)
    outf = outg.transpose(1, 0, 2).reshape(NP, H)

    W1p = jnp.pad(W1, ((0, 0), (0, 128 - C)))
    b1p = jnp.pad(b1, (0, 128 - C), constant_values=-1e30).reshape(1, 128)
    ls = pl.pallas_call(
        _head_body,
        grid=(10,),
        in_specs=[
            pl.BlockSpec((N // 10, H), lambda i: (i, 0)),
            pl.BlockSpec((H, 128), lambda i: (0, 0)),
            pl.BlockSpec((1, 128), lambda i: (0, 0)),
        ],
        out_specs=pl.BlockSpec((N // 10, 128), lambda i: (i, 0)),
        out_shape=jax.ShapeDtypeStruct((N, 128), jnp.float32),
    )(outf[:N], W1p, b1p)
    return ls[:, :C]
